# baseline (speedup 1.0000x reference)
import jax
import jax.numpy as jnp
from jax.experimental import pallas as pl
from jax.experimental.pallas import tpu as pltpu

_VC = 1733
_NC = 29
_U = 8


def _body(sidx_ref, order_ref, starts_ref, w_ref, out_ref, sem):
    c = pl.program_id(0)
    n0 = starts_ref[c]
    n1 = starts_ref[c + 1]
    cnt = n1 - n0
    base = c * _VC

    def issue1(k):
        r = sidx_ref[k] - base
        p = order_ref[k]
        pltpu.make_async_copy(
            w_ref.at[pl.ds(r, 1)],
            out_ref.at[pl.ds(p, 1)],
            sem,
        ).start()

    def issue_u(j, carry):
        k0 = n0 + j * _U
        for u in range(_U):
            issue1(k0 + u)
        return carry

    nu = cnt // _U
    jax.lax.fori_loop(0, nu, issue_u, 0)

    def issue_rem(k, carry):
        issue1(k)
        return carry

    jax.lax.fori_loop(n0 + nu * _U, n1, issue_rem, 0)

    @pl.when(cnt > 0)
    def _wait():
        pltpu.make_async_copy(
            out_ref.at[pl.ds(0, cnt)],
            out_ref.at[pl.ds(0, cnt)],
            sem,
        ).wait()


def kernel(input, weight):
    bptt, bsize = input.shape
    vocab, emsize = weight.shape
    n = bptt * bsize
    idx = input.reshape(n).astype(jnp.int32)
    order = jnp.argsort(idx).astype(jnp.int32)
    sidx = jnp.take(idx, order)
    bounds = (jnp.arange(_NC + 1, dtype=jnp.int32) * _VC)
    starts = jnp.searchsorted(sidx, bounds).astype(jnp.int32)
    w3 = weight.reshape(vocab, 1, emsize)
    out = pl.pallas_call(
        _body,
        grid_spec=pltpu.PrefetchScalarGridSpec(
            num_scalar_prefetch=3,
            grid=(_NC,),
            in_specs=[pl.BlockSpec((_VC, 1, emsize),
                                   lambda c, s, o, st: (c, 0, 0))],
            out_specs=pl.BlockSpec(memory_space=pl.ANY),
            scratch_shapes=[pltpu.SemaphoreType.DMA],
        ),
        out_shape=jax.ShapeDtypeStruct((n, 1, emsize), weight.dtype),
        compiler_params=pltpu.CompilerParams(
            dimension_semantics=("arbitrary",)),
        name="embedding_stream_scatter",
    )(sidx, order, starts, w3)
    return out.reshape(bptt, bsize, emsize)
